# baseline (speedup 1.0000x reference)
import functools

import jax
import jax.numpy as jnp
from jax import lax
from jax.experimental import pallas as pl
from jax.experimental.pallas import tpu as pltpu
from jax.experimental.pallas import tpu_sc as plsc

N = 10000
E = 320000
D_IN = 128
D_HID = 128
D_OUT = 40
D_PAD = 128

NC = 2
NS = 16
NW = NC * NS
CH = 80
EPW = E // NW
NCHUNK = EPW // CH

ZR = 640
ZR_LAST = N - 15 * ZR


def _make_scatter(D):
    mesh = plsc.VectorSubcoreMesh(core_axis_name="c", subcore_axis_name="s")

    @functools.partial(
        pl.kernel,
        out_type=jax.ShapeDtypeStruct((NC, N, D), jnp.float32),
        mesh=mesh,
        scratch_types=(
            [pltpu.VMEM((CH,), jnp.int32)] * 6 +
            [pltpu.VMEM((CH,), jnp.int32)] * 6 +
            [pltpu.VMEM((CH, D), jnp.float32)] * 4 +
            [pltpu.VMEM_SHARED((N, D), jnp.float32)] +
            [pltpu.SemaphoreType.DMA] * 14
        ),
    )
    def scatter_kernel(h_hbm, src_hbm, dst_hbm, zero_hbm, out_hbm,
                       s0, s1, s2, s3, s4, s5, d0, d1, d2, d3, d4, d5,
                       r0, r1, r2, r3, acc_sh,
                       g0, g1, g2, g3, t0, t1, t2, t3,
                       i0, i1, i2, i3, i4, i5):
        c = lax.axis_index("c")
        s = lax.axis_index("s")
        wid = s * NC + c

        @pl.when(s < 15)
        def _():
            pltpu.sync_copy(zero_hbm.at[pl.ds(s * ZR, ZR)],
                            acc_sh.at[pl.ds(s * ZR, ZR)])

        @pl.when(s == 15)
        def _():
            pltpu.sync_copy(zero_hbm.at[pl.ds(15 * ZR, ZR_LAST)],
                            acc_sh.at[pl.ds(15 * ZR, ZR_LAST)])

        plsc.subcore_barrier()

        src = (s0, s1, s2, s3, s4, s5)
        dst = (d0, d1, d2, d3, d4, d5)
        rows = (r0, r1, r2, r3)
        gsem = (g0, g1, g2, g3)
        ssem = (t0, t1, t2, t3)
        isem = (i0, i1, i2, i3, i4, i5)
        base0 = wid * EPW

        def idx_issue(i, b):
            base = base0 + jnp.minimum(i, NCHUNK - 1) * CH
            pltpu.async_copy(src_hbm.at[pl.ds(base, CH)], src[b], isem[b])
            pltpu.async_copy(dst_hbm.at[pl.ds(base, CH)], dst[b], isem[b])

        def iwait(b):
            pltpu.make_async_copy(src_hbm.at[pl.ds(0, CH)], src[b],
                                  isem[b]).wait()
            pltpu.make_async_copy(dst_hbm.at[pl.ds(0, CH)], dst[b],
                                  isem[b]).wait()

        def gwait(b):
            pltpu.make_async_copy(h_hbm.at[pl.ds(0, CH)], rows[b],
                                  gsem[b]).wait()

        def swait(b4, b6):
            pltpu.make_async_copy(rows[b4], acc_sh.at[dst[b6]],
                                  ssem[b4]).wait()

        def body(i, q4, q6, pre_swait, pre_gather, pre_idx):
            if pre_swait:
                swait((q4 + 2) % 4, (q6 + 4) % 6)
            if pre_gather:
                iwait((q6 + 2) % 6)
                pltpu.async_copy(h_hbm.at[src[(q6 + 2) % 6]],
                                 rows[(q4 + 2) % 4],
                                 gsem[(q4 + 2) % 4])
            gwait(q4)
            pltpu.async_copy(rows[q4], acc_sh.at[dst[q6]], ssem[q4],
                             add=True)
            if pre_idx:
                idx_issue(i + 4, (q6 + 4) % 6)

        pltpu.sync_copy(src_hbm.at[pl.ds(base0, CH)], s0)
        pltpu.sync_copy(dst_hbm.at[pl.ds(base0, CH)], d0)
        pltpu.sync_copy(src_hbm.at[pl.ds(base0 + CH, CH)], s1)
        pltpu.sync_copy(dst_hbm.at[pl.ds(base0 + CH, CH)], d1)
        pltpu.async_copy(h_hbm.at[s0], r0, g0)
        pltpu.async_copy(h_hbm.at[s1], r1, g1)
        idx_issue(2, 2)
        idx_issue(3, 3)

        body(0, 0, 0, pre_swait=False, pre_gather=True, pre_idx=True)
        body(1, 1, 1, pre_swait=False, pre_gather=True, pre_idx=True)

        MAIN_LO, MAIN_HI = 2, 2 + 12 * ((NCHUNK - 3 - 2 + 1) // 12)

        @pl.loop(MAIN_LO, MAIN_HI, step=12)
        def _(j):
            for b in range(12):
                body(j + b, (MAIN_LO + b) % 4, (MAIN_LO + b) % 6,
                     pre_swait=True, pre_gather=True, pre_idx=True)

        for t in range(MAIN_HI, NCHUNK):
            body(t, t % 4, t % 6, pre_swait=True,
                 pre_gather=(t <= NCHUNK - 3), pre_idx=(t <= NCHUNK - 5))

        swait((NCHUNK - 2) % 4, (NCHUNK - 2) % 6)
        swait((NCHUNK - 1) % 4, (NCHUNK - 1) % 6)
        iwait(NCHUNK % 6)

        plsc.subcore_barrier()

        @pl.when(s < 15)
        def _():
            pltpu.sync_copy(acc_sh.at[pl.ds(s * ZR, ZR)],
                            out_hbm.at[c].at[pl.ds(s * ZR, ZR)])

        @pl.when(s == 15)
        def _():
            pltpu.sync_copy(acc_sh.at[pl.ds(15 * ZR, ZR_LAST)],
                            out_hbm.at[c].at[pl.ds(15 * ZR, ZR_LAST)])

    return scatter_kernel


_scatter128 = _make_scatter(D_HID)


def _mm(x, w):
    def body(x_ref, w_ref, o_ref):
        o_ref[...] = jnp.dot(x_ref[...], w_ref[...],
                             preferred_element_type=jnp.float32)

    return pl.pallas_call(
        body,
        out_shape=jax.ShapeDtypeStruct((x.shape[0], w.shape[1]), jnp.float32),
    )(x, w)


def _fuse_mm(p, b, w):
    def body(p_ref, b_ref, w_ref, o_ref):
        h = jnp.maximum(p_ref[0] + p_ref[1] + b_ref[...], 0.0)
        o_ref[...] = jnp.dot(h, w_ref[...], preferred_element_type=jnp.float32)

    return pl.pallas_call(
        body,
        out_shape=jax.ShapeDtypeStruct((p.shape[1], w.shape[1]), jnp.float32),
    )(p, b.reshape(1, -1), w)


def _finalize(q, b2p):
    def body(q_ref, b_ref, o_ref):
        h = q_ref[0] + q_ref[1] + b_ref[...]
        m = jnp.max(h, axis=1, keepdims=True)
        e = jnp.exp(h - m)
        lse = jnp.log(jnp.sum(e, axis=1, keepdims=True))
        o_ref[...] = h - m - lse

    return pl.pallas_call(
        body,
        out_shape=jax.ShapeDtypeStruct((q.shape[1], q.shape[2]), jnp.float32),
    )(q, b2p.reshape(1, -1))


def kernel(x, edge_index, W0, b0, W1, b1, W2, b2):
    src = edge_index[0]
    dst2 = edge_index[1]
    zeros128 = jnp.zeros((N, D_HID), jnp.float32)
    W2p = jnp.pad(W2, ((0, 0), (0, D_PAD - D_OUT)))
    b2p = jnp.concatenate(
        [b2, jnp.full((D_PAD - D_OUT,), -1e30, jnp.float32)])

    h = _mm(x, W0)
    p = _scatter128(h, src, dst2, zeros128)
    h = _fuse_mm(p, b0, W1)
    p = _scatter128(h, src, dst2, zeros128)
    h = _fuse_mm(p, b1, W2p)
    q = _scatter128(h, src, dst2, zeros128)
    out = _finalize(q, b2p)
    return out[:, :D_OUT]


# speedup vs baseline: 1.0462x; 1.0462x over previous
import functools

import jax
import jax.numpy as jnp
from jax import lax
from jax.experimental import pallas as pl
from jax.experimental.pallas import tpu as pltpu
from jax.experimental.pallas import tpu_sc as plsc

N = 10000
E = 320000
D_IN = 128
D_HID = 128
D_OUT = 40
D_PAD = 128

NC = 2
NS = 16
NW = NC * NS
CH = 80
EPW = E // NW
NCHUNK = EPW // CH

ZR = 640
ZR_LAST = N - 15 * ZR


def _make_scatter(D):
    mesh = plsc.VectorSubcoreMesh(core_axis_name="c", subcore_axis_name="s")

    @functools.partial(
        pl.kernel,
        out_type=jax.ShapeDtypeStruct((NC, N, D), jnp.float32),
        mesh=mesh,
        scratch_types=(
            [pltpu.VMEM((CH,), jnp.int32)] * 6 +
            [pltpu.VMEM((CH,), jnp.int32)] * 6 +
            [pltpu.VMEM((CH, D), jnp.float32)] * 4 +
            [pltpu.VMEM_SHARED((N, D), jnp.float32)] +
            [pltpu.SemaphoreType.DMA] * 10
        ),
    )
    def scatter_kernel(h_hbm, src_hbm, dst_hbm, zero_hbm, out_hbm,
                       s0, s1, s2, s3, s4, s5, d0, d1, d2, d3, d4, d5,
                       r0, r1, r2, r3, acc_sh,
                       g0, g1, g2, g3, i0, i1, i2, i3, i4, i5):
        c = lax.axis_index("c")
        s = lax.axis_index("s")
        wid = s * NC + c

        @pl.when(s < 15)
        def _():
            pltpu.sync_copy(zero_hbm.at[pl.ds(s * ZR, ZR)],
                            acc_sh.at[pl.ds(s * ZR, ZR)])

        @pl.when(s == 15)
        def _():
            pltpu.sync_copy(zero_hbm.at[pl.ds(15 * ZR, ZR_LAST)],
                            acc_sh.at[pl.ds(15 * ZR, ZR_LAST)])

        plsc.subcore_barrier()

        src = (s0, s1, s2, s3, s4, s5)
        dst = (d0, d1, d2, d3, d4, d5)
        rows = (r0, r1, r2, r3)
        gsem = (g0, g1, g2, g3)
        isem = (i0, i1, i2, i3, i4, i5)
        base0 = wid * EPW

        def idx_issue(i, b):
            base = base0 + jnp.minimum(i, NCHUNK - 1) * CH
            pltpu.async_copy(src_hbm.at[pl.ds(base, CH)], src[b], isem[b])
            pltpu.async_copy(dst_hbm.at[pl.ds(base, CH)], dst[b], isem[b])

        def iwait(b):
            pltpu.make_async_copy(src_hbm.at[pl.ds(0, CH)], src[b],
                                  isem[b]).wait()
            pltpu.make_async_copy(dst_hbm.at[pl.ds(0, CH)], dst[b],
                                  isem[b]).wait()

        def gwait(b):
            pltpu.make_async_copy(h_hbm.at[pl.ds(0, CH)], rows[b],
                                  gsem[b]).wait()

        def body(i, q4, q6, pre_swait, pre_gather, pre_idx):
            if pre_gather:
                iwait((q6 + 2) % 6)
                pltpu.async_copy(h_hbm.at[src[(q6 + 2) % 6]],
                                 rows[(q4 + 2) % 4],
                                 gsem[(q4 + 2) % 4])
            gwait(q4)
            pltpu.sync_copy(rows[q4], acc_sh.at[dst[q6]], add=True)
            if pre_idx:
                idx_issue(i + 4, (q6 + 4) % 6)

        pltpu.sync_copy(src_hbm.at[pl.ds(base0, CH)], s0)
        pltpu.sync_copy(dst_hbm.at[pl.ds(base0, CH)], d0)
        pltpu.sync_copy(src_hbm.at[pl.ds(base0 + CH, CH)], s1)
        pltpu.sync_copy(dst_hbm.at[pl.ds(base0 + CH, CH)], d1)
        pltpu.async_copy(h_hbm.at[s0], r0, g0)
        pltpu.async_copy(h_hbm.at[s1], r1, g1)
        idx_issue(2, 2)
        idx_issue(3, 3)

        body(0, 0, 0, pre_swait=False, pre_gather=True, pre_idx=True)
        body(1, 1, 1, pre_swait=False, pre_gather=True, pre_idx=True)

        MAIN_LO, MAIN_HI = 2, 2 + 12 * ((NCHUNK - 3 - 2 + 1) // 12)

        @pl.loop(MAIN_LO, MAIN_HI, step=12)
        def _(j):
            for b in range(12):
                body(j + b, (MAIN_LO + b) % 4, (MAIN_LO + b) % 6,
                     pre_swait=True, pre_gather=True, pre_idx=True)

        for t in range(MAIN_HI, NCHUNK):
            body(t, t % 4, t % 6, pre_swait=True,
                 pre_gather=(t <= NCHUNK - 3), pre_idx=(t <= NCHUNK - 5))

        iwait(NCHUNK % 6)

        plsc.subcore_barrier()

        @pl.when(s < 15)
        def _():
            pltpu.sync_copy(acc_sh.at[pl.ds(s * ZR, ZR)],
                            out_hbm.at[c].at[pl.ds(s * ZR, ZR)])

        @pl.when(s == 15)
        def _():
            pltpu.sync_copy(acc_sh.at[pl.ds(15 * ZR, ZR_LAST)],
                            out_hbm.at[c].at[pl.ds(15 * ZR, ZR_LAST)])

    return scatter_kernel


_scatter128 = _make_scatter(D_HID)


def _mm(x, w):
    def body(x_ref, w_ref, o_ref):
        o_ref[...] = jnp.dot(x_ref[...], w_ref[...],
                             preferred_element_type=jnp.float32)

    return pl.pallas_call(
        body,
        out_shape=jax.ShapeDtypeStruct((x.shape[0], w.shape[1]), jnp.float32),
    )(x, w)


def _fuse_mm(p, b, w):
    def body(p_ref, b_ref, w_ref, o_ref):
        h = jnp.maximum(p_ref[0] + p_ref[1] + b_ref[...], 0.0)
        o_ref[...] = jnp.dot(h, w_ref[...], preferred_element_type=jnp.float32)

    return pl.pallas_call(
        body,
        out_shape=jax.ShapeDtypeStruct((p.shape[1], w.shape[1]), jnp.float32),
    )(p, b.reshape(1, -1), w)


def _finalize(q, b2p):
    def body(q_ref, b_ref, o_ref):
        h = q_ref[0] + q_ref[1] + b_ref[...]
        m = jnp.max(h, axis=1, keepdims=True)
        e = jnp.exp(h - m)
        lse = jnp.log(jnp.sum(e, axis=1, keepdims=True))
        o_ref[...] = h - m - lse

    return pl.pallas_call(
        body,
        out_shape=jax.ShapeDtypeStruct((q.shape[1], q.shape[2]), jnp.float32),
    )(q, b2p.reshape(1, -1))


def kernel(x, edge_index, W0, b0, W1, b1, W2, b2):
    src = edge_index[0]
    dst2 = edge_index[1]
    zeros128 = jnp.zeros((N, D_HID), jnp.float32)
    W2p = jnp.pad(W2, ((0, 0), (0, D_PAD - D_OUT)))
    b2p = jnp.concatenate(
        [b2, jnp.full((D_PAD - D_OUT,), -1e30, jnp.float32)])

    h = _mm(x, W0)
    p = _scatter128(h, src, dst2, zeros128)
    h = _fuse_mm(p, b0, W1)
    p = _scatter128(h, src, dst2, zeros128)
    h = _fuse_mm(p, b1, W2p)
    q = _scatter128(h, src, dst2, zeros128)
    out = _finalize(q, b2p)
    return out[:, :D_OUT]
